# baseline (speedup 1.0000x reference)
import jax
import jax.numpy as jnp
from jax.experimental import pallas as pl
from jax.experimental.pallas import tpu as pltpu

_K = 8
_R = 512


def _body(x_vmem, x_smem, y_ref, out_ref):
    b = pl.program_id(0)
    sub = _R // 128
    labels = x_vmem[pl.ds(b * sub, sub), :]
    n_bad = jnp.sum(((labels < 0) | (labels >= _K)).astype(jnp.int32))

    out_ref[:, :] = y_ref[:, :]

    @pl.when(n_bad > 0)
    def _fixup():
        def zero_bad_row(i, _):
            lab = x_smem[b * _R + i]

            @pl.when((lab < 0) | (lab >= _K))
            def _z():
                out_ref[pl.ds(i, 1), :] = jnp.zeros((1, out_ref.shape[1]),
                                                    out_ref.dtype)
            return _
        jax.lax.fori_loop(0, _R, zero_bad_row, 0)


def kernel(x, y):
    n, d = y.shape
    grid = n // _R
    xi = x.astype(jnp.int32)
    x2 = xi.reshape(n // 128, 128)
    return pl.pallas_call(
        _body,
        grid=(grid,),
        in_specs=[
            pl.BlockSpec((n // 128, 128), lambda i: (0, 0)),
            pl.BlockSpec(memory_space=pltpu.MemorySpace.SMEM),
            pl.BlockSpec((_R, d), lambda i: (i, 0)),
        ],
        out_specs=pl.BlockSpec((_R, d), lambda i: (i, 0)),
        out_shape=jax.ShapeDtypeStruct((n, d), y.dtype),
        compiler_params=pltpu.CompilerParams(
            dimension_semantics=("arbitrary",),
        ),
    )(x2, xi, y)


# speedup vs baseline: 1.0214x; 1.0214x over previous
import jax
import jax.numpy as jnp
from jax.experimental import pallas as pl
from jax.experimental.pallas import tpu as pltpu

_K = 8
_R = 1024


def _body(x_vmem, x_smem, y_ref, out_ref):
    b = pl.program_id(0)
    sub = _R // 128
    labels = x_vmem[pl.ds(b * sub, sub), :]
    n_bad = jnp.sum(((labels < 0) | (labels >= _K)).astype(jnp.int32))

    out_ref[:, :] = y_ref[:, :]

    @pl.when(n_bad > 0)
    def _fixup():
        def zero_bad_row(i, _):
            lab = x_smem[b * _R + i]

            @pl.when((lab < 0) | (lab >= _K))
            def _z():
                out_ref[pl.ds(i, 1), :] = jnp.zeros((1, out_ref.shape[1]),
                                                    out_ref.dtype)
            return _
        jax.lax.fori_loop(0, _R, zero_bad_row, 0)


def kernel(x, y):
    n, d = y.shape
    grid = n // _R
    xi = x.astype(jnp.int32)
    x2 = xi.reshape(n // 128, 128)
    return pl.pallas_call(
        _body,
        grid=(grid,),
        in_specs=[
            pl.BlockSpec((n // 128, 128), lambda i: (0, 0)),
            pl.BlockSpec(memory_space=pltpu.MemorySpace.SMEM),
            pl.BlockSpec((_R, d), lambda i: (i, 0)),
        ],
        out_specs=pl.BlockSpec((_R, d), lambda i: (i, 0)),
        out_shape=jax.ShapeDtypeStruct((n, d), y.dtype),
        compiler_params=pltpu.CompilerParams(
            dimension_semantics=("arbitrary",),
        ),
    )(x2, xi, y)
